# Initial kernel scaffold; baseline (speedup 1.0000x reference)
import jax
import jax.numpy as jnp
from jax.experimental import pallas as pl


def kernel(x, edge_index, batch, W1, b1, W2, b2, Wh, bh):
    raise NotImplementedError("write your pallas kernel here")


# speedup vs baseline: 23.0204x; 23.0204x over previous
import functools

import jax
import jax.numpy as jnp
from jax import lax
from jax.experimental import pallas as pl
from jax.experimental.pallas import tpu as pltpu
from jax.experimental.pallas import tpu_sc as plsc

N = 10000
F = 128
H = 64
G = 64
E = 320000

NC, NS = 2, 16
NW = NC * NS
EBLK = 128
NP = 10112
ZR = NP // NS
DUMMY = N

ETOT = E + N
NB = -(-ETOT // (NW * EBLK))
EP = NW * EBLK * NB
EPW = NB * EBLK

def _sc_deg_body(dst_hbm, ones_hbm, z_hbm, out_hbm, dstv, onesv, zv, acc):
    cid = lax.axis_index("c")
    sid = lax.axis_index("s")
    wid = cid * NS + sid
    zbase = sid * ZR
    pltpu.sync_copy(z_hbm, zv)
    pltpu.sync_copy(zv, acc.at[pl.ds(zbase, ZR)])
    pltpu.sync_copy(dst_hbm.at[wid], dstv)
    pltpu.sync_copy(ones_hbm, onesv)
    plsc.subcore_barrier()

    def body(j, carry):
        pltpu.sync_copy(onesv, acc.at[dstv.at[j]], add=True)
        return carry

    lax.fori_loop(0, NB, body, 0, unroll=False)
    plsc.subcore_barrier()
    pltpu.sync_copy(acc.at[pl.ds(zbase, ZR)], out_hbm.at[cid, pl.ds(zbase, ZR)])


def _sc_agg_body(table_hbm, src_hbm, dst_hbm, z_hbm, out_hbm,
                 srcv, dstv, rows, zv, acc, sem):
    cid = lax.axis_index("c")
    sid = lax.axis_index("s")
    wid = cid * NS + sid
    zbase = sid * ZR
    pltpu.sync_copy(z_hbm, zv)
    pltpu.sync_copy(zv, acc.at[pl.ds(zbase, ZR)])
    pltpu.sync_copy(src_hbm.at[wid], srcv)
    pltpu.sync_copy(dst_hbm.at[wid], dstv)
    plsc.subcore_barrier()

    def body(j, carry):
        pltpu.async_copy(table_hbm.at[srcv.at[j]], rows, sem).wait()
        pltpu.sync_copy(rows, acc.at[dstv.at[j]], add=True)
        return carry

    lax.fori_loop(0, NB, body, 0, unroll=False)
    plsc.subcore_barrier()
    pltpu.sync_copy(acc.at[pl.ds(zbase, ZR)], out_hbm.at[cid, pl.ds(zbase, ZR)])


@functools.lru_cache(maxsize=1)
def _sc_kernels():
    mesh = plsc.VectorSubcoreMesh(
        core_axis_name="c", subcore_axis_name="s",
        num_cores=NC, num_subcores=NS)
    params = pltpu.CompilerParams(use_tc_tiling_on_sc=False)
    sc_deg = pl.kernel(
        _sc_deg_body,
        out_type=jax.ShapeDtypeStruct((NC, NP, 16), jnp.float32),
        mesh=mesh,
        compiler_params=params,
        scratch_types=[
            pltpu.VMEM((NB, EBLK), jnp.int32),
            pltpu.VMEM((EBLK, 16), jnp.float32),
            pltpu.VMEM((ZR, 16), jnp.float32),
            pltpu.VMEM_SHARED((NP, 16), jnp.float32),
        ],
    )
    sc_agg = pl.kernel(
        _sc_agg_body,
        out_type=jax.ShapeDtypeStruct((NC, NP, H), jnp.float32),
        mesh=mesh,
        compiler_params=params,
        scratch_types=[
            pltpu.VMEM((NB, EBLK), jnp.int32),
            pltpu.VMEM((NB, EBLK), jnp.int32),
            pltpu.VMEM((EBLK, H), jnp.float32),
            pltpu.VMEM((ZR, H), jnp.float32),
            pltpu.VMEM_SHARED((NP, H), jnp.float32),
            pltpu.SemaphoreType.DMA,
        ],
    )
    return sc_deg, sc_agg


def _tc_prep1_body(x_ref, w1_ref, degp_ref, hn_ref, dinv_ref):
    deg = degp_ref[0, :, 0:1] + degp_ref[1, :, 0:1]
    dinv = jnp.where(deg > 0.0, lax.rsqrt(deg), 0.0)
    h = jnp.dot(x_ref[...], w1_ref[...], preferred_element_type=jnp.float32)
    hn_ref[...] = h * dinv
    dinv_ref[...] = dinv


_tc_prep1 = pl.pallas_call(
    _tc_prep1_body,
    out_shape=(jax.ShapeDtypeStruct((NP, H), jnp.float32),
               jax.ShapeDtypeStruct((NP, 1), jnp.float32)),
)


def _tc_prep2_body(p_ref, dinv_ref, b1_ref, w2_ref, hn_ref):
    dinv = dinv_ref[...]
    t = (p_ref[0] + p_ref[1]) * dinv + b1_ref[...]
    t = jnp.maximum(t, 0.0)
    hn_ref[...] = jnp.dot(t, w2_ref[...],
                          preferred_element_type=jnp.float32) * dinv


_tc_prep2 = pl.pallas_call(
    _tc_prep2_body,
    out_shape=jax.ShapeDtypeStruct((NP, H), jnp.float32),
)


def _tc_head_body(p_ref, dinv_ref, b2_ref, batch_ref, wh_ref, bh_ref, out_ref):
    h2 = (p_ref[0] + p_ref[1]) * dinv_ref[...] + b2_ref[...]
    h2 = jnp.maximum(h2, 0.0)
    b = batch_ref[...]
    gids = lax.broadcasted_iota(jnp.int32, (G, 1), 0)
    mask = (b == gids).astype(jnp.float32)
    sums = jnp.dot(mask, h2, preferred_element_type=jnp.float32)
    cnt = jnp.sum(mask, axis=1, keepdims=True)
    pooled = sums / jnp.maximum(cnt, 1.0)
    out_ref[...] = (jnp.dot(pooled, wh_ref[...],
                            preferred_element_type=jnp.float32) + bh_ref[...])


_tc_head = pl.pallas_call(
    _tc_head_body,
    out_shape=jax.ShapeDtypeStruct((G, 1), jnp.float32),
)


def kernel(x, edge_index, batch, W1, b1, W2, b2, Wh, bh):
    f32 = jnp.float32
    loop = jnp.arange(N, dtype=jnp.int32)
    pad = jnp.full((EP - ETOT,), DUMMY, jnp.int32)
    src3 = jnp.concatenate([edge_index[0].astype(jnp.int32), loop, pad]
                           ).reshape(NW, NB, EBLK)
    dst3 = jnp.concatenate([edge_index[1].astype(jnp.int32), loop, pad]
                           ).reshape(NW, NB, EBLK)

    ones16 = jnp.ones((EBLK, 16), f32)
    z16 = jnp.zeros((ZR, 16), f32)
    zH = jnp.zeros((ZR, H), f32)
    xp = jnp.zeros((NP, F), f32).at[:N].set(x)
    batchp = jnp.full((1, NP), G, jnp.int32).at[0, :N].set(batch)

    sc_deg, sc_agg = _sc_kernels()
    degp = sc_deg(dst3, ones16, z16)
    hn1, dinv = _tc_prep1(xp, W1, degp)
    p1 = sc_agg(hn1, src3, dst3, zH)
    hn2 = _tc_prep2(p1, dinv, b1.reshape(1, H), W2)
    p2 = sc_agg(hn2, src3, dst3, zH)
    out = _tc_head(p2, dinv, b2.reshape(1, H), batchp,
                   Wh, bh.reshape(1, 1))
    return out.reshape(G)
